# baseline (speedup 1.0000x reference)
import jax
import jax.numpy as jnp
from jax import lax
from jax.experimental import pallas as pl
from jax.experimental.pallas import tpu as pltpu
from jax.experimental.pallas import tpu_sc as plsc

_N = 16384
_D = 4096
_E = 64
_K = 8
_T = 1024
_Z_LOSS_W = 0.001
_AUX_W = 0.01

_NC = 2
_NS = 16
_NW = _NC * _NS
_TPW = _N // _NW
_L = 16


def _gate_body(x_ref, w_ref, lt_ref, z_ref, p_ref):
    x = x_ref[...]
    w = w_ref[...]
    lt = jax.lax.dot_general(w, x, (((1,), (1,)), ((), ())),
                             preferred_element_type=jnp.float32)
    lt_ref[...] = lt
    m = jnp.max(lt, axis=0, keepdims=True)
    e = jnp.exp(lt - m)
    s = jnp.sum(e, axis=0, keepdims=True)
    lse = m + jnp.log(s)
    z_ref[...] = jnp.sum(lse * lse, axis=(0, 1), keepdims=True).reshape(1, 1, 1)
    p_ref[...] = jnp.sum(e / s, axis=1, keepdims=True).reshape(1, 1, _E)


def _gate_stage(x, W):
    grid = _N // _T
    return pl.pallas_call(
        _gate_body,
        grid=(grid,),
        in_specs=[
            pl.BlockSpec((_T, _D), lambda i: (i, 0)),
            pl.BlockSpec((_E, _D), lambda i: (0, 0)),
        ],
        out_specs=[
            pl.BlockSpec((_E, _T), lambda i: (0, i)),
            pl.BlockSpec((1, 1, 1), lambda i: (i, 0, 0)),
            pl.BlockSpec((1, 1, _E), lambda i: (i, 0, 0)),
        ],
        out_shape=[
            jax.ShapeDtypeStruct((_E, _N), jnp.float32),
            jax.ShapeDtypeStruct((grid, 1, 1), jnp.float32),
            jax.ShapeDtypeStruct((grid, 1, _E), jnp.float32),
        ],
        compiler_params=pltpu.CompilerParams(
            dimension_semantics=("parallel",),
        ),
    )(x, W)


def _topk_body(lt_hbm, idx_hbm, rw_hbm, buf_v, idxo_v, rwo_v, sem):
    wid = lax.axis_index("s") * _NC + lax.axis_index("c")
    base = wid * _TPW
    copies = [
        pltpu.make_async_copy(
            lt_hbm.at[e, pl.ds(base, _TPW)],
            buf_v.at[pl.ds(e * _TPW, _TPW)],
            sem,
        )
        for e in range(_E)
    ]
    for c in copies:
        c.start()
    for c in copies:
        c.wait()

    lane = lax.iota(jnp.int32, _L)
    neg = jnp.float32(-jnp.inf)

    def one_group(g, carry):
        t0 = g * _L
        keys = [jnp.full((_L,), neg, dtype=jnp.float32) for _ in range(_K)]
        idxs = [jnp.full((_L,), 0, dtype=jnp.int32) for _ in range(_K)]

        def one_expert(e, st):
            ks = list(st[:_K])
            is_ = list(st[_K:])
            nk = buf_v[pl.ds(e * _TPW + t0, _L)]
            ni = jnp.full((_L,), e, dtype=jnp.int32)
            for j in range(_K):
                gt = nk > ks[j]
                ks_j, is_j = ks[j], is_[j]
                ks[j] = jnp.where(gt, nk, ks_j)
                is_[j] = jnp.where(gt, ni, is_j)
                nk = jnp.where(gt, ks_j, nk)
                ni = jnp.where(gt, is_j, ni)
            return tuple(ks) + tuple(is_)

        st = lax.fori_loop(0, _E, one_expert, tuple(keys) + tuple(idxs))
        ks = st[:_K]
        is_ = st[_K:]
        evs = [jnp.exp(k - ks[0]) for k in ks]
        stot = evs[0]
        for ev in evs[1:]:
            stot = stot + ev
        rinv = jnp.float32(1.0) / stot
        for j in range(_K):
            idxo_v[pl.ds(j * _TPW + t0, _L)] = is_[j]
            rwo_v[pl.ds(j * _TPW + t0, _L)] = evs[j] * rinv
        return carry

    lax.fori_loop(0, _TPW // _L, one_group, jnp.int32(0))
    for j in range(_K):
        pltpu.sync_copy(idxo_v.at[pl.ds(j * _TPW, _TPW)],
                        idx_hbm.at[j, pl.ds(base, _TPW)])
        pltpu.sync_copy(rwo_v.at[pl.ds(j * _TPW, _TPW)],
                        rw_hbm.at[j, pl.ds(base, _TPW)])


def _topk_stage(lt):
    mesh = plsc.VectorSubcoreMesh(core_axis_name="c", subcore_axis_name="s")
    f = pl.kernel(
        _topk_body,
        mesh=mesh,
        out_type=[
            jax.ShapeDtypeStruct((_K, _N), jnp.int32),
            jax.ShapeDtypeStruct((_K, _N), jnp.float32),
        ],
        scratch_types=[
            pltpu.VMEM((_E * _TPW,), jnp.float32),
            pltpu.VMEM((_TPW * _K,), jnp.int32),
            pltpu.VMEM((_TPW * _K,), jnp.float32),
            pltpu.SemaphoreType.DMA,
        ],
    )
    return f(lt)


def kernel(x, W):
    lt, zp, pp = _gate_stage(x, W)
    idx_kn, rw_kn = _topk_stage(lt)
    z_loss = (jnp.sum(zp) / _N) * _Z_LOSS_W
    lb_loss = _AUX_W * _AUX_W * (jnp.sum(pp) / _N - float(_K))
    return (idx_kn.T, rw_kn.T, (z_loss + lb_loss).astype(jnp.float32))
